# baseline (speedup 1.0000x reference)
import functools

import jax
import jax.numpy as jnp
from jax.experimental import pallas as pl
from jax.experimental.pallas import tpu as pltpu

_TILE = 8192


def _matmul_body(state_ref, w_ref, out_ref):
    out_ref[...] = jax.lax.dot_general(
        state_ref[...],
        w_ref[...],
        (((1,), (0,)), ((), ())),
        preferred_element_type=jnp.float32,
    )


@jax.jit
def kernel(state, W):
    b, d = state.shape
    d2, v = W.shape
    grid = (pl.cdiv(v, _TILE),)
    return pl.pallas_call(
        _matmul_body,
        grid=grid,
        in_specs=[
            pl.BlockSpec((b, d), lambda i: (0, 0)),
            pl.BlockSpec((d, _TILE), lambda i: (0, i)),
        ],
        out_specs=pl.BlockSpec((b, _TILE), lambda i: (0, i)),
        out_shape=jax.ShapeDtypeStruct((b, v), jnp.float32),
        compiler_params=pltpu.CompilerParams(
            dimension_semantics=("arbitrary",),
        ),
    )(state, W)


# speedup vs baseline: 1.4946x; 1.4946x over previous
import functools

import jax
import jax.numpy as jnp
from jax.experimental import pallas as pl
from jax.experimental.pallas import tpu as pltpu

_TILE = 32768


def _matmul_body(state_ref, w_ref, out_ref):
    out_ref[...] = jax.lax.dot_general(
        state_ref[...],
        w_ref[...],
        (((1,), (0,)), ((), ())),
        preferred_element_type=jnp.float32,
    )


@jax.jit
def kernel(state, W):
    b, d = state.shape
    d2, v = W.shape
    grid = (pl.cdiv(v, _TILE),)
    return pl.pallas_call(
        _matmul_body,
        grid=grid,
        in_specs=[
            pl.BlockSpec((b, d), lambda i: (0, 0)),
            pl.BlockSpec((d, _TILE), lambda i: (0, i)),
        ],
        out_specs=pl.BlockSpec((b, _TILE), lambda i: (0, i)),
        out_shape=jax.ShapeDtypeStruct((b, v), jnp.float32),
        compiler_params=pltpu.CompilerParams(
            dimension_semantics=("arbitrary",),
        ),
    )(state, W)
